# baseline (speedup 1.0000x reference)
import functools

import jax
import jax.numpy as jnp
from jax import lax
from jax.experimental import pallas as pl
from jax.experimental.pallas import tpu as pltpu
from jax.experimental.pallas import tpu_sc as plsc

BATCH = 16
MAX_LEN = 2048
TOTAL_CAP = BATCH * MAX_LEN

L = 16
CHUNK = MAX_LEN
WIN = CHUNK + 8
WIN_BASE_MAX = TOTAL_CAP - WIN


def _tok_body(flat_hbm, cu_hbm, out_hbm, cu_v, win_v, out_v):
    wid = lax.axis_index("s")
    b = wid
    c0 = 0

    pltpu.sync_copy(cu_hbm, cu_v.at[pl.ds(0, BATCH + 1)])

    lanes = lax.iota(jnp.int32, L)
    cu_pair = cu_v[pl.ds(b, L)]
    start_s = cu_pair[0]
    end_s = cu_pair[1]
    row_len = end_s - start_s

    win_start = start_s + c0
    base8 = jnp.minimum(win_start & ~7, WIN_BASE_MAX)
    base8 = pl.multiple_of(base8, 8)
    pltpu.sync_copy(flat_hbm.at[pl.ds(base8, WIN)], win_v.at[pl.ds(0, WIN)])
    off = win_start - base8

    @plsc.parallel_loop(0, CHUNK, step=L, unroll=16)
    def body(p):
        pos = p + lanes
        vals = win_v[pl.ds(off + p, L)]
        out_v[pl.ds(p, L)] = jnp.where(pos < row_len, vals, 0.0)

    pltpu.sync_copy(out_v, out_hbm.at[b])


@jax.jit
def _tokpad(flat_values, cu_seqlens):
    mesh = plsc.VectorSubcoreMesh(
        core_axis_name="c", subcore_axis_name="s", num_cores=1)
    k = functools.partial(
        pl.kernel,
        mesh=mesh,
        out_type=jax.ShapeDtypeStruct((BATCH, CHUNK), jnp.float32),
        scratch_types=[
            pltpu.VMEM((128,), jnp.int32),
            pltpu.VMEM((WIN + CHUNK + 2048 + 64,), jnp.float32),
            pltpu.VMEM((CHUNK,), jnp.float32),
        ],
    )(_tok_body)
    return k(flat_values, cu_seqlens)


def kernel(flat_values, cu_seqlens):
    out = _tokpad(flat_values, cu_seqlens.astype(jnp.int32))
    return out.reshape(BATCH, MAX_LEN)


# speedup vs baseline: 1.0119x; 1.0119x over previous
import functools

import jax
import jax.numpy as jnp
from jax import lax
from jax.experimental import pallas as pl
from jax.experimental.pallas import tpu as pltpu
from jax.experimental.pallas import tpu_sc as plsc

BATCH = 16
MAX_LEN = 2048
TOTAL_CAP = BATCH * MAX_LEN

L = 16
CHUNK = MAX_LEN
WIN = CHUNK + 8
WIN_BASE_MAX = TOTAL_CAP - WIN


def _tok_body(flat_hbm, cu_hbm, out_hbm, cu_v, win_v, out_v):
    wid = lax.axis_index("s")
    b = wid
    c0 = 0

    pltpu.sync_copy(cu_hbm, cu_v.at[pl.ds(0, BATCH + 1)])

    lanes = lax.iota(jnp.int32, L)
    cu_pair = cu_v[pl.ds(b, L)]
    start_s = cu_pair[0]
    end_s = cu_pair[1]
    row_len = end_s - start_s

    win_start = start_s + c0
    base8 = jnp.minimum(win_start & ~7, WIN_BASE_MAX)
    base8 = pl.multiple_of(base8, 8)
    pltpu.sync_copy(flat_hbm.at[pl.ds(base8, WIN)], win_v.at[pl.ds(0, WIN)])
    off = win_start - base8

    @plsc.parallel_loop(0, CHUNK, step=L, unroll=8)
    def body(p):
        pos = p + lanes
        vals = win_v[pl.ds(off + p, L)]
        out_v[pl.ds(p, L)] = jnp.where(pos < row_len, vals, 0.0)

    pltpu.sync_copy(out_v, out_hbm.at[b])


@jax.jit
def _tokpad(flat_values, cu_seqlens):
    mesh = plsc.VectorSubcoreMesh(
        core_axis_name="c", subcore_axis_name="s", num_cores=1)
    k = functools.partial(
        pl.kernel,
        mesh=mesh,
        out_type=jax.ShapeDtypeStruct((BATCH, CHUNK), jnp.float32),
        scratch_types=[
            pltpu.VMEM((128,), jnp.int32),
            pltpu.VMEM((WIN + CHUNK + 2048 + 64,), jnp.float32),
            pltpu.VMEM((CHUNK,), jnp.float32),
        ],
    )(_tok_body)
    return k(flat_values, cu_seqlens)


def kernel(flat_values, cu_seqlens):
    out = _tokpad(flat_values, cu_seqlens.astype(jnp.int32))
    return out.reshape(BATCH, MAX_LEN)
